# Initial kernel scaffold; baseline (speedup 1.0000x reference)
import functools

import jax
import jax.numpy as jnp
from jax import lax
from jax.experimental import pallas as pl
from jax.experimental.pallas import tpu as pltpu
from jax.experimental.pallas import tpu_sc as plsc

F32 = jnp.float32


def _leaky(x):
    return jnp.where(x > 0, x, 0.01 * x)


def _k1_body(nf, wr, wc, b1, p_ref, q_ref):
    x = nf[...]
    p_ref[...] = jnp.dot(x, wr[...], preferred_element_type=F32) + b1[...]
    q_ref[...] = jnp.dot(x, wc[...], preferred_element_type=F32)


def _node_precompute(node_feat, We1_r, We1_c, be1_2d, N, D, H, NB):
    return pl.pallas_call(
        _k1_body,
        grid=(N // NB,),
        in_specs=[
            pl.BlockSpec((NB, D), lambda i: (i, 0)),
            pl.BlockSpec((D, H), lambda i: (0, 0)),
            pl.BlockSpec((D, H), lambda i: (0, 0)),
            pl.BlockSpec((1, H), lambda i: (0, 0)),
        ],
        out_specs=[
            pl.BlockSpec((NB, H), lambda i: (i, 0)),
            pl.BlockSpec((NB, H), lambda i: (i, 0)),
        ],
        out_shape=[
            jax.ShapeDtypeStruct((N, H), F32),
            jax.ShapeDtypeStruct((N, H), F32),
        ],
    )(node_feat, We1_r, We1_c, be1_2d)


def _edge_gather(P, Q, C16, row2d, col2d, E, H, W2):
    mesh = plsc.VectorSubcoreMesh(
        core_axis_name="core", subcore_axis_name="subcore")

    @functools.partial(
        pl.kernel,
        out_type=(
            jax.ShapeDtypeStruct((E, H), F32),
            jax.ShapeDtypeStruct((E, H), F32),
            jax.ShapeDtypeStruct((E, 16), F32),
            jax.ShapeDtypeStruct((E, 16), F32),
        ),
        mesh=mesh,
    )
    def k2(p_hbm, q_hbm, c_hbm, row_hbm, col_hbm,
           gp_hbm, gq_hbm, cr_hbm, cc_hbm):
        def body(row_v, col_v, gp_v, gq_v, cr_v, cc_v):
            pltpu.sync_copy(p_hbm.at[row_v.at[0]], gp_v)
            pltpu.sync_copy(q_hbm.at[col_v.at[0]], gq_v)
            pltpu.sync_copy(c_hbm.at[row_v.at[0]], cr_v)
            pltpu.sync_copy(c_hbm.at[col_v.at[0]], cc_v)

        pltpu.emit_pipeline(
            body,
            grid=(E // W2,),
            in_specs=[
                pl.BlockSpec((1, W2), lambda i: (0, i)),
                pl.BlockSpec((1, W2), lambda i: (0, i)),
            ],
            out_specs=[
                pl.BlockSpec((W2, H), lambda i: (i, 0)),
                pl.BlockSpec((W2, H), lambda i: (i, 0)),
                pl.BlockSpec((W2, 16), lambda i: (i, 0)),
                pl.BlockSpec((W2, 16), lambda i: (i, 0)),
            ],
            core_axis_name=("core", "subcore"),
            dimension_semantics=(pltpu.PARALLEL,),
        )(row_hbm, col_hbm, gp_hbm, gq_hbm, cr_hbm, cc_hbm)

    return k2(P, Q, C16, row2d, col2d)


def _k3_body(gp, gq, cr, cc, ef, w1e, wrad, w2, b2, wc1, bc1, wc2,
             m_ref, tr_ref):
    df = cr[...] - cc[...]
    dx = df[:, 0:1]
    dy = df[:, 1:2]
    radial = dx * dx + dy * dy
    pre1 = (gp[...] + gq[...] + radial * wrad[...]
            + jnp.dot(ef[...], w1e[...], preferred_element_type=F32))
    x1 = _leaky(pre1)
    m = _leaky(jnp.dot(x1, w2[...], preferred_element_type=F32) + b2[...])
    t = _leaky(jnp.dot(m, wc1[...], preferred_element_type=F32) + bc1[...])
    s = jnp.sum(t * wc2[...], axis=1, keepdims=True)
    inv = 1.0 / (jnp.sqrt(radial) + 1e-8)
    m_ref[...] = m
    tr_ref[...] = df * (s * inv)


def _edge_mlp(GP, GQ, CR, CC, edge_feat, W1e, wrad2d, We2, be2_2d,
              Wc1, bc1_2d, wc2_2d, E, DE, H, EB):
    full = lambda a, b: pl.BlockSpec((a, b), lambda i: (0, 0))
    blk = lambda b: pl.BlockSpec((EB, b), lambda i: (i, 0))
    return pl.pallas_call(
        _k3_body,
        grid=(E // EB,),
        in_specs=[
            blk(H), blk(H), blk(16), blk(16), blk(DE),
            full(DE, H), full(1, H), full(H, H), full(1, H),
            full(H, H), full(1, H), full(1, H),
        ],
        out_specs=[blk(H), blk(16)],
        out_shape=[
            jax.ShapeDtypeStruct((E, H), F32),
            jax.ShapeDtypeStruct((E, 16), F32),
        ],
    )(GP, GQ, CR, CC, edge_feat, W1e, wrad2d, We2, be2_2d, Wc1, bc1_2d,
      wc2_2d)


def _scatter_add(m, trans, row_flat, N, E, H):
    NH = N // 2
    RPT = (NH + 15) // 16
    NHP = RPT * 16
    LAST = NH - 15 * RPT
    E16 = E // 16
    CH = 80
    NCH = E16 // CH
    mesh = plsc.VectorSubcoreMesh(
        core_axis_name="core", subcore_axis_name="subcore")
    z1 = jnp.zeros((NHP, H), F32)
    z2 = jnp.zeros((NHP, 16), F32)

    @functools.partial(
        pl.kernel,
        out_type=(
            jax.ShapeDtypeStruct((N, H), F32),
            jax.ShapeDtypeStruct((N, 16), F32),
        ),
        mesh=mesh,
        scratch_types=[
            pltpu.VMEM((CH, H), F32),
            pltpu.VMEM((CH, 16), F32),
            pltpu.VMEM((CH,), jnp.int32),
            pltpu.VMEM((CH,), jnp.int32),
            pltpu.VMEM_SHARED((NHP, H), F32),
            pltpu.VMEM_SHARED((NHP, 16), F32),
        ],
    )
    def k4(m_hbm, tr_hbm, row_hbm, z1_hbm, z2_hbm, agg_hbm, aggc_hbm,
           m_v, t_v, idx_v, loc_v, aggS, aggcS):
        c = lax.axis_index("core")
        s = lax.axis_index("subcore")
        base = c * NH
        pltpu.sync_copy(z1_hbm.at[pl.ds(s * RPT, RPT)],
                        aggS.at[pl.ds(s * RPT, RPT)])
        pltpu.sync_copy(z2_hbm.at[pl.ds(s * RPT, RPT)],
                        aggcS.at[pl.ds(s * RPT, RPT)])
        plsc.subcore_barrier()

        @pl.loop(0, NCH)
        def _(k):
            off = s * E16 + k * CH
            pltpu.sync_copy(row_hbm.at[pl.ds(off, CH)], idx_v)

            @pl.loop(0, CH // 16)
            def _(j):
                iv = idx_v[pl.ds(j * 16, 16)]
                lv = iv - base
                ok = (lv >= 0) & (lv < NH)
                loc_v[pl.ds(j * 16, 16)] = jnp.where(ok, lv, NH)

            pltpu.sync_copy(m_hbm.at[pl.ds(off, CH)], m_v)
            pltpu.sync_copy(tr_hbm.at[pl.ds(off, CH)], t_v)
            pltpu.sync_copy(m_v, aggS.at[loc_v], add=True)
            pltpu.sync_copy(t_v, aggcS.at[loc_v], add=True)

        plsc.subcore_barrier()

        @pl.when(s < 15)
        def _():
            pltpu.sync_copy(aggS.at[pl.ds(s * RPT, RPT)],
                            agg_hbm.at[pl.ds(base + s * RPT, RPT)])
            pltpu.sync_copy(aggcS.at[pl.ds(s * RPT, RPT)],
                            aggc_hbm.at[pl.ds(base + s * RPT, RPT)])

        @pl.when(s == 15)
        def _():
            pltpu.sync_copy(aggS.at[pl.ds(15 * RPT, LAST)],
                            agg_hbm.at[pl.ds(base + 15 * RPT, LAST)])
            pltpu.sync_copy(aggcS.at[pl.ds(15 * RPT, LAST)],
                            aggc_hbm.at[pl.ds(base + 15 * RPT, LAST)])

    return k4(m, trans, row_flat, z1, z2)


def _k5_body(nf, agg, c16, aggc, wa, wb, b1, w2, b2, hp_ref, co_ref,
             sums_ref):
    i = pl.program_id(0)
    h1 = _leaky(jnp.dot(nf[...], wa[...], preferred_element_type=F32)
                + jnp.dot(agg[...], wb[...], preferred_element_type=F32)
                + b1[...])
    hp = jnp.dot(h1, w2[...], preferred_element_type=F32) + b2[...]
    co = c16[...] + aggc[...]
    hp_ref[...] = hp
    co_ref[...] = co

    @pl.when(i == 0)
    def _():
        sums_ref[...] = jnp.zeros_like(sums_ref)

    D = hp.shape[1]
    pad = ((0, 0), (0, D - co.shape[1]))
    sums_ref[0:1, :] += jnp.sum(hp, axis=0, keepdims=True)
    sums_ref[1:2, :] += jnp.sum(hp * hp, axis=0, keepdims=True)
    sums_ref[2:3, :] += jnp.pad(jnp.sum(co, axis=0, keepdims=True), pad)
    sums_ref[3:4, :] += jnp.pad(jnp.sum(co * co, axis=0, keepdims=True), pad)


def _node_mlp(node_feat, agg, C16, aggc, Wn1a, Wn1b, bn1_2d, Wn2, bn2_2d,
              N, D, H, NB):
    full = lambda a, b: pl.BlockSpec((a, b), lambda i: (0, 0))
    blk = lambda b: pl.BlockSpec((NB, b), lambda i: (i, 0))
    return pl.pallas_call(
        _k5_body,
        grid=(N // NB,),
        in_specs=[
            blk(D), blk(H), blk(16), blk(16),
            full(D, H), full(H, H), full(1, H), full(H, D), full(1, D),
        ],
        out_specs=[blk(D), blk(16), pl.BlockSpec((8, D), lambda i: (0, 0))],
        out_shape=[
            jax.ShapeDtypeStruct((N, D), F32),
            jax.ShapeDtypeStruct((N, 16), F32),
            jax.ShapeDtypeStruct((8, D), F32),
        ],
    )(node_feat, agg, C16, aggc, Wn1a, Wn1b, bn1_2d, Wn2, bn2_2d)


def _k6_body(n_inv_ref, hp, co, sums, g1, b1, g2, b2, h_ref, c_ref):
    n_inv = n_inv_ref[0]
    srow = sums[...]
    mean1 = srow[0:1, :] * n_inv
    var1 = srow[1:2, :] * n_inv - mean1 * mean1
    inv1 = lax.rsqrt(var1 + 1e-5)
    h = (hp[...] - mean1) * inv1 * g1[...] + b1[...]
    h_ref[...] = _leaky(h)
    mean2 = srow[2:3, 0:16] * n_inv
    var2 = srow[3:4, 0:16] * n_inv - mean2 * mean2
    inv2 = lax.rsqrt(var2 + 1e-5)
    c_ref[...] = (co[...] - mean2) * inv2 * g2[...] + b2[...]


def _bn_apply(hp, co16, sums, g1_2d, b1_2d, g2_16, b2_16, N, D, NB):
    full = lambda a, b: pl.BlockSpec((a, b), lambda i: (0, 0))
    blk = lambda b: pl.BlockSpec((NB, b), lambda i: (i, 0))
    n_inv = jnp.full((1,), 1.0 / N, F32)
    return pl.pallas_call(
        _k6_body,
        grid=(N // NB,),
        in_specs=[
            pl.BlockSpec(memory_space=pltpu.SMEM),
            blk(D), blk(16), full(8, D),
            full(1, D), full(1, D), full(1, 16), full(1, 16),
        ],
        out_specs=[blk(D), blk(16)],
        out_shape=[
            jax.ShapeDtypeStruct((N, D), F32),
            jax.ShapeDtypeStruct((N, 16), F32),
        ],
    )(n_inv, hp, co16, sums, g1_2d, b1_2d, g2_16, b2_16)


def kernel(coords, node_feat, edge_feat, edge_index, batch_index,
           num_sampled_nodes_per_hop, num_sampled_edges_per_hop,
           We1, be1, We2, be2, Wn1, bn1, Wn2, bn2,
           Wc1, bc1, Wc2, gamma1, beta1, gamma2, beta2):
    N, CD = coords.shape
    E, DE = edge_feat.shape
    D = node_feat.shape[1]
    H = We2.shape[0]

    We1_r = We1[:D]
    We1_c = We1[D:2 * D]
    wrad2d = We1[2 * D:2 * D + 1]
    W1e = We1[2 * D + 1:]

    C16 = jnp.pad(coords, ((0, 0), (0, 16 - CD)))
    row2d = edge_index[0:1]
    col2d = edge_index[1:2]
    row_flat = edge_index[0]

    P, Q = _node_precompute(node_feat, We1_r, We1_c, be1.reshape(1, H),
                            N, D, H, NB=1000)
    GP, GQ, CR, CC = _edge_gather(P, Q, C16, row2d, col2d, E, H, W2=128)
    m, trans = _edge_mlp(GP, GQ, CR, CC, edge_feat, W1e, wrad2d,
                         We2, be2.reshape(1, H), Wc1, bc1.reshape(1, H),
                         Wc2.reshape(1, H), E, DE, H, EB=640)
    agg, aggc = _scatter_add(m, trans, row_flat, N, E, H)
    hp, co16, sums = _node_mlp(node_feat, agg, C16, aggc,
                               Wn1[:D], Wn1[D:], bn1.reshape(1, H),
                               Wn2, bn2.reshape(1, D), N, D, H, NB=1000)
    g2_16 = jnp.pad(gamma2, (0, 16 - CD)).reshape(1, 16)
    b2_16 = jnp.pad(beta2, (0, 16 - CD)).reshape(1, 16)
    h, c16 = _bn_apply(hp, co16, sums, gamma1.reshape(1, D),
                       beta1.reshape(1, D), g2_16, b2_16, N, D, NB=1000)
    return (h, c16[:, :CD], edge_feat)


# speedup vs baseline: 1.3586x; 1.3586x over previous
import dataclasses
import functools

import jax
import jax.numpy as jnp
from jax import lax
from jax.experimental import pallas as pl
from jax.experimental.pallas import tpu as pltpu
from jax.experimental.pallas import tpu_sc as plsc

F32 = jnp.float32


def _sc_params():
    cp = pltpu.CompilerParams()
    if "needs_layout_passes" in pltpu.CompilerParams.__dataclass_fields__:
        cp = dataclasses.replace(cp, needs_layout_passes=False)
    return cp


def _leaky(x):
    return jnp.where(x > 0, x, 0.01 * x)


def _k1_body(nf, wr, wc, b1, p_ref, q_ref):
    x = nf[...]
    p_ref[...] = jnp.dot(x, wr[...], preferred_element_type=F32) + b1[...]
    q_ref[...] = jnp.dot(x, wc[...], preferred_element_type=F32)


def _node_precompute(node_feat, We1_r, We1_c, be1_2d, N, D, H, NB):
    return pl.pallas_call(
        _k1_body,
        grid=(N // NB,),
        in_specs=[
            pl.BlockSpec((NB, D), lambda i: (i, 0)),
            pl.BlockSpec((D, H), lambda i: (0, 0)),
            pl.BlockSpec((D, H), lambda i: (0, 0)),
            pl.BlockSpec((1, H), lambda i: (0, 0)),
        ],
        out_specs=[
            pl.BlockSpec((NB, H), lambda i: (i, 0)),
            pl.BlockSpec((NB, H), lambda i: (i, 0)),
        ],
        out_shape=[
            jax.ShapeDtypeStruct((N, H), F32),
            jax.ShapeDtypeStruct((N, H), F32),
        ],
    )(node_feat, We1_r, We1_c, be1_2d)


def _edge_gather(P, Q, cflat, row2d, col2d, N, E, H, W2):
    mesh = plsc.VectorSubcoreMesh(
        core_axis_name="core", subcore_axis_name="subcore",
        num_cores=2, num_subcores=16)

    @functools.partial(
        pl.kernel,
        out_type=(
            jax.ShapeDtypeStruct((E, H), F32),
            jax.ShapeDtypeStruct((E, 16), F32),
        ),
        mesh=mesh,
        scratch_types=[pltpu.VMEM((2 * N,), F32)],
        compiler_params=_sc_params(),
    )
    def k2a(p_hbm, c_hbm, row_hbm, col_hbm, g_hbm, geo_hbm, coords_v):
        pltpu.sync_copy(c_hbm.at[0], coords_v)
        zeros16 = jnp.zeros((16,), jnp.int32)
        ones16 = jnp.full((16,), 1, jnp.int32)
        twos16 = jnp.full((16,), 2, jnp.int32)

        def body(row_v, col_v, g_v, geo_v):
            pltpu.sync_copy(p_hbm.at[row_v.at[0]], g_v)

            @pl.loop(0, W2 // 16)
            def _(j):
                r16 = row_v[0, pl.ds(j * 16, 16)]
                c16 = col_v[0, pl.ds(j * 16, 16)]
                cxr = plsc.load_gather(coords_v, [r16 * 2])
                cyr = plsc.load_gather(coords_v, [r16 * 2 + 1])
                cxc = plsc.load_gather(coords_v, [c16 * 2])
                cyc = plsc.load_gather(coords_v, [c16 * 2 + 1])
                dx = cxr - cxc
                dy = cyr - cyc
                rad = dx * dx + dy * dy
                rows = j * 16 + lax.iota(jnp.int32, 16)
                plsc.store_scatter(geo_v, [rows, zeros16], dx)
                plsc.store_scatter(geo_v, [rows, ones16], dy)
                plsc.store_scatter(geo_v, [rows, twos16], rad)

        pltpu.emit_pipeline(
            body,
            grid=(E // W2,),
            in_specs=[
                pl.BlockSpec((1, W2), lambda i: (0, i)),
                pl.BlockSpec((1, W2), lambda i: (0, i)),
            ],
            out_specs=[
                pl.BlockSpec((W2, H), lambda i: (i, 0)),
                pl.BlockSpec((W2, 16), lambda i: (i, 0)),
            ],
            core_axis_name=("core", "subcore"),
            dimension_semantics=(pltpu.PARALLEL,),
        )(row_hbm, col_hbm, g_hbm, geo_hbm)

    @functools.partial(
        pl.kernel,
        out_type=jax.ShapeDtypeStruct((E, H), F32),
        mesh=mesh,
        compiler_params=_sc_params(),
    )
    def k2b(q_hbm, col_hbm, gq_hbm):
        def body(col_v, gq_v):
            pltpu.sync_copy(q_hbm.at[col_v.at[0]], gq_v)

        pltpu.emit_pipeline(
            body,
            grid=(E // W2,),
            in_specs=[pl.BlockSpec((1, W2), lambda i: (0, i))],
            out_specs=[pl.BlockSpec((W2, H), lambda i: (i, 0))],
            core_axis_name=("core", "subcore"),
            dimension_semantics=(pltpu.PARALLEL,),
        )(col_hbm, gq_hbm)

    GP, GEO = k2a(P, cflat, row2d, col2d)
    GQ = k2b(Q, col2d)
    return GP, GQ, GEO


def _k3_body(gp, gq, geo, ef, w1e, wrad, w2, b2, wc1, bc1, wc2, mx_ref):
    g = geo[...]
    radial = g[:, 2:3]
    pre1 = (gp[...] + gq[...] + radial * wrad[...]
            + jnp.dot(ef[...], w1e[...], preferred_element_type=F32))
    x1 = _leaky(pre1)
    m = _leaky(jnp.dot(x1, w2[...], preferred_element_type=F32) + b2[...])
    t = _leaky(jnp.dot(m, wc1[...], preferred_element_type=F32) + bc1[...])
    s = jnp.sum(t * wc2[...], axis=1, keepdims=True)
    inv = 1.0 / (jnp.sqrt(radial) + 1e-8)
    H = m.shape[1]
    mx_ref[:, 0:H] = m
    mx_ref[:, H:H + 16] = g * (s * inv)


def _edge_mlp(GP, GQ, GEO, edge_feat, W1e, wrad2d, We2, be2_2d,
              Wc1, bc1_2d, wc2_2d, E, DE, H, EB):
    full = lambda a, b: pl.BlockSpec((a, b), lambda i: (0, 0))
    blk = lambda b: pl.BlockSpec((EB, b), lambda i: (i, 0))
    return pl.pallas_call(
        _k3_body,
        grid=(E // EB,),
        in_specs=[
            blk(H), blk(H), blk(16), blk(DE),
            full(DE, H), full(1, H), full(H, H), full(1, H),
            full(H, H), full(1, H), full(1, H),
        ],
        out_specs=[blk(H + 128)],
        out_shape=[jax.ShapeDtypeStruct((E, H + 128), F32)],
    )(GP, GQ, GEO, edge_feat, W1e, wrad2d, We2, be2_2d, Wc1, bc1_2d,
      wc2_2d)[0]


def _k4_body(row_s, mxb, agg_ref, *, EB2):
    i = pl.program_id(0)

    @pl.when(i == 0)
    def _():
        agg_ref[...] = jnp.zeros_like(agg_ref)

    def step(e, carry):
        loc = row_s[0, 0, e]
        agg_ref[pl.ds(loc, 1), :] += mxb[pl.ds(e, 1), :]
        return carry

    lax.fori_loop(0, EB2, step, 0)


def _scatter_add(mx, row_flat, N, E, HX):
    NP = N + 16
    EB2 = 2000
    row3d = row_flat.reshape(E // EB2, 1, EB2)
    return pl.pallas_call(
        functools.partial(_k4_body, EB2=EB2),
        grid=(E // EB2,),
        in_specs=[
            pl.BlockSpec((1, 1, EB2), lambda i: (i, 0, 0),
                         memory_space=pltpu.SMEM),
            pl.BlockSpec((EB2, HX), lambda i: (i, 0)),
        ],
        out_specs=pl.BlockSpec((NP, HX), lambda i: (0, 0)),
        out_shape=jax.ShapeDtypeStruct((NP, HX), F32),
    )(row3d, mx)


def _k5_body(nf, agg, c16, aggc, wa, wb, b1, w2, b2, hp_ref, co_ref,
             sums_ref):
    i = pl.program_id(0)
    h1 = _leaky(jnp.dot(nf[...], wa[...], preferred_element_type=F32)
                + jnp.dot(agg[...], wb[...], preferred_element_type=F32)
                + b1[...])
    hp = jnp.dot(h1, w2[...], preferred_element_type=F32) + b2[...]
    co = c16[...] + aggc[...]
    hp_ref[...] = hp
    co_ref[...] = co

    @pl.when(i == 0)
    def _():
        sums_ref[...] = jnp.zeros_like(sums_ref)

    D = hp.shape[1]
    pad = ((0, 0), (0, D - co.shape[1]))
    sums_ref[0:1, :] += jnp.sum(hp, axis=0, keepdims=True)
    sums_ref[1:2, :] += jnp.sum(hp * hp, axis=0, keepdims=True)
    sums_ref[2:3, :] += jnp.pad(jnp.sum(co, axis=0, keepdims=True), pad)
    sums_ref[3:4, :] += jnp.pad(jnp.sum(co * co, axis=0, keepdims=True), pad)


def _node_mlp(node_feat, agg, C16, aggc, Wn1a, Wn1b, bn1_2d, Wn2, bn2_2d,
              N, D, H, NB):
    full = lambda a, b: pl.BlockSpec((a, b), lambda i: (0, 0))
    blk = lambda b: pl.BlockSpec((NB, b), lambda i: (i, 0))
    return pl.pallas_call(
        _k5_body,
        grid=(N // NB,),
        in_specs=[
            blk(D), blk(H), blk(16), blk(16),
            full(D, H), full(H, H), full(1, H), full(H, D), full(1, D),
        ],
        out_specs=[blk(D), blk(16), pl.BlockSpec((8, D), lambda i: (0, 0))],
        out_shape=[
            jax.ShapeDtypeStruct((N, D), F32),
            jax.ShapeDtypeStruct((N, 16), F32),
            jax.ShapeDtypeStruct((8, D), F32),
        ],
    )(node_feat, agg, C16, aggc, Wn1a, Wn1b, bn1_2d, Wn2, bn2_2d)


def _k6_body(n_inv_ref, hp, co, sums, g1, b1, g2, b2, h_ref, c_ref):
    n_inv = n_inv_ref[0]
    srow = sums[...]
    mean1 = srow[0:1, :] * n_inv
    var1 = srow[1:2, :] * n_inv - mean1 * mean1
    inv1 = lax.rsqrt(var1 + 1e-5)
    h = (hp[...] - mean1) * inv1 * g1[...] + b1[...]
    h_ref[...] = _leaky(h)
    mean2 = srow[2:3, 0:16] * n_inv
    var2 = srow[3:4, 0:16] * n_inv - mean2 * mean2
    inv2 = lax.rsqrt(var2 + 1e-5)
    c_ref[...] = (co[...] - mean2) * inv2 * g2[...] + b2[...]


def _bn_apply(hp, co16, sums, g1_2d, b1_2d, g2_16, b2_16, N, D, NB):
    full = lambda a, b: pl.BlockSpec((a, b), lambda i: (0, 0))
    blk = lambda b: pl.BlockSpec((NB, b), lambda i: (i, 0))
    n_inv = jnp.full((1,), 1.0 / N, F32)
    return pl.pallas_call(
        _k6_body,
        grid=(N // NB,),
        in_specs=[
            pl.BlockSpec(memory_space=pltpu.SMEM),
            blk(D), blk(16), full(8, D),
            full(1, D), full(1, D), full(1, 16), full(1, 16),
        ],
        out_specs=[blk(D), blk(16)],
        out_shape=[
            jax.ShapeDtypeStruct((N, D), F32),
            jax.ShapeDtypeStruct((N, 16), F32),
        ],
    )(n_inv, hp, co16, sums, g1_2d, b1_2d, g2_16, b2_16)


def kernel(coords, node_feat, edge_feat, edge_index, batch_index,
           num_sampled_nodes_per_hop, num_sampled_edges_per_hop,
           We1, be1, We2, be2, Wn1, bn1, Wn2, bn2,
           Wc1, bc1, Wc2, gamma1, beta1, gamma2, beta2):
    N, CD = coords.shape
    E, DE = edge_feat.shape
    D = node_feat.shape[1]
    H = We2.shape[0]

    We1_r = We1[:D]
    We1_c = We1[D:2 * D]
    wrad2d = We1[2 * D:2 * D + 1]
    W1e = We1[2 * D + 1:]

    C16 = jnp.pad(coords, ((0, 0), (0, 16 - CD)))
    cflat = coords.reshape(1, 2 * N)
    row2d = edge_index[0:1]
    col2d = edge_index[1:2]
    row_flat = edge_index[0]

    P, Q = _node_precompute(node_feat, We1_r, We1_c, be1.reshape(1, H),
                            N, D, H, NB=1000)
    GP, GQ, GEO = _edge_gather(P, Q, cflat, row2d, col2d, N, E, H, W2=128)
    mx = _edge_mlp(GP, GQ, GEO, edge_feat, W1e, wrad2d,
                   We2, be2.reshape(1, H), Wc1, bc1.reshape(1, H),
                   Wc2.reshape(1, H), E, DE, H, EB=640)
    agg_f = _scatter_add(mx, row_flat, N, E, H + 128)
    agg = agg_f[:N, 0:H]
    aggc = agg_f[:N, H:H + 16]
    hp, co16, sums = _node_mlp(node_feat, agg, C16, aggc,
                               Wn1[:D], Wn1[D:], bn1.reshape(1, H),
                               Wn2, bn2.reshape(1, D), N, D, H, NB=1000)
    g2_16 = jnp.pad(gamma2, (0, 16 - CD)).reshape(1, 16)
    b2_16 = jnp.pad(beta2, (0, 16 - CD)).reshape(1, 16)
    h, c16 = _bn_apply(hp, co16, sums, gamma1.reshape(1, D),
                       beta1.reshape(1, D), g2_16, b2_16, N, D, NB=1000)
    return (h, c16[:, :CD], edge_feat)
